# baseline (speedup 1.0000x reference)
import functools

import jax
import jax.numpy as jnp
from jax import lax
from jax.experimental import pallas as pl
from jax.experimental.pallas import tpu as pltpu
from jax.experimental.pallas import tpu_sc as plsc

NC = 2
NS = 16
L = 16
CH = 128


def _node_pre_body(h_ref, wa_ref, wb_ref, ba_ref, a_ref, b_ref):
    h = h_ref[...]
    a_ref[...] = jnp.dot(h, wa_ref[...], preferred_element_type=jnp.float32) + ba_ref[...]
    b_ref[...] = jnp.dot(h, wb_ref[...], preferred_element_type=jnp.float32)


def _node_pre(h, we1a_t, we1b_t, be1, bn):
    n, d = h.shape
    return pl.pallas_call(
        _node_pre_body,
        grid=(n // bn,),
        in_specs=[
            pl.BlockSpec((bn, d), lambda i: (i, 0)),
            pl.BlockSpec((d, d), lambda i: (0, 0)),
            pl.BlockSpec((d, d), lambda i: (0, 0)),
            pl.BlockSpec((1, d), lambda i: (0, 0)),
        ],
        out_specs=[
            pl.BlockSpec((bn, d), lambda i: (i, 0)),
            pl.BlockSpec((bn, d), lambda i: (i, 0)),
        ],
        out_shape=[
            jax.ShapeDtypeStruct((n, d), jnp.float32),
            jax.ShapeDtypeStruct((n, d), jnp.float32),
        ],
    )(h, we1a_t, we1b_t, be1)


def _sc_gather(a, b, x0, x1, x2, ling, colg, e_pad, kch, base):
    n, d = a.shape
    mesh = plsc.VectorSubcoreMesh(core_axis_name="c", subcore_axis_name="s")

    @functools.partial(
        pl.kernel,
        out_type=[
            jax.ShapeDtypeStruct((e_pad, d), jnp.float32),
            jax.ShapeDtypeStruct((e_pad,), jnp.float32),
        ],
        mesh=mesh,
        scratch_types=[
            pltpu.VMEM((kch * CH,), jnp.int32),
            pltpu.VMEM((kch * CH,), jnp.int32),
            pltpu.VMEM((CH, d), jnp.float32),
            pltpu.VMEM((CH, d), jnp.float32),
            pltpu.VMEM((CH, d), jnp.float32),
            pltpu.VMEM((CH, d), jnp.float32),
            pltpu.VMEM((CH,), jnp.float32),
            pltpu.VMEM((CH,), jnp.float32),
            pltpu.VMEM((n,), jnp.float32),
            pltpu.VMEM((n,), jnp.float32),
            pltpu.VMEM((n,), jnp.float32),
            pltpu.SemaphoreType.DMA,
            pltpu.SemaphoreType.DMA,
            pltpu.SemaphoreType.DMA,
            pltpu.SemaphoreType.DMA,
        ],
        compiler_params=pltpu.CompilerParams(needs_layout_passes=False),
    )
    def k(a_hbm, b_hbm, x0_hbm, x1_hbm, x2_hbm, lin_hbm, col_hbm,
          pre_hbm, drf_hbm,
          linw, colw, bufa0, bufa1, bufb0, bufb1,
          drb0, drb1, x0v, x1v, x2v, si0, si1, so0, so1):
        wid = lax.axis_index("s") * NC + lax.axis_index("c")
        row0 = base + wid * kch
        rloc0 = wid * kch
        bufa = (bufa0, bufa1)
        bufb = (bufb0, bufb1)
        drb = (drb0, drb1)
        s_in = (si0, si1)
        s_out = (so0, so1)
        pltpu.sync_copy(x0_hbm, x0v)
        pltpu.sync_copy(x1_hbm, x1v)
        pltpu.sync_copy(x2_hbm, x2v)

        pltpu.sync_copy(lin_hbm.at[pl.ds(row0 * CH, kch * CH)], linw)
        pltpu.sync_copy(col_hbm.at[pl.ds(row0 * CH, kch * CH)], colw)
        pltpu.async_copy(a_hbm.at[linw.at[pl.ds(0, CH)]], bufa[0], s_in[0])
        pltpu.async_copy(b_hbm.at[colw.at[pl.ds(0, CH)]], bufb[0], s_in[0])

        @pl.loop(0, kch, step=4)
        def _blk(j4):
            for jj in range(4):
                par = jj & 1
                c = j4 + jj
                pltpu.make_async_copy(a_hbm.at[linw.at[pl.ds(0, CH)]],
                                      bufa[par], s_in[par]).wait()
                pltpu.make_async_copy(b_hbm.at[colw.at[pl.ds(0, CH)]],
                                      bufb[par], s_in[par]).wait()

                for g in range(CH // L):
                    sl = pl.ds(c * CH + g * L, L)
                    li = linw[sl]
                    ci = colw[sl]
                    dx = plsc.load_gather(x0v, [li]) - plsc.load_gather(x0v, [ci])
                    dy = plsc.load_gather(x1v, [li]) - plsc.load_gather(x1v, [ci])
                    dz = plsc.load_gather(x2v, [li]) - plsc.load_gather(x2v, [ci])
                    drb[par][pl.ds(g * L, L)] = dx * dx + dy * dy + dz * dz

                @pl.when(c >= 1)
                def _():
                    pltpu.make_async_copy(bufb[1 - par],
                                          pre_hbm.at[pl.ds(0, CH)],
                                          s_out[1 - par]).wait()
                    pltpu.make_async_copy(drb[1 - par],
                                          drf_hbm.at[pl.ds(0, CH)],
                                          s_out[1 - par]).wait()

                @pl.when(c + 1 < kch)
                def _():
                    pltpu.async_copy(
                        a_hbm.at[linw.at[pl.ds((c + 1) * CH, CH)]],
                        bufa[1 - par], s_in[1 - par])
                    pltpu.async_copy(
                        b_hbm.at[colw.at[pl.ds((c + 1) * CH, CH)]],
                        bufb[1 - par], s_in[1 - par])

                ba = bufa[par]
                bb = bufb[par]

                @pl.loop(0, CH, unroll=4)
                def _row(i):
                    for cc in range(d // L):
                        sl = pl.ds(cc * L, L)
                        bb[i, sl] = ba[i, sl] + bb[i, sl]

                r = rloc0 + c
                pltpu.async_copy(bb, pre_hbm.at[pl.ds(r * CH, CH)], s_out[par])
                pltpu.async_copy(drb[par], drf_hbm.at[pl.ds(r * CH, CH)],
                                 s_out[par])

        pltpu.make_async_copy(bufb[1], pre_hbm.at[pl.ds(0, CH)],
                              s_out[1]).wait()
        pltpu.make_async_copy(drb[1], drf_hbm.at[pl.ds(0, CH)],
                              s_out[1]).wait()

    return k(a, b, x0, x1, x2, ling, colg)


def _edge_body(pre_ref, dr_ref, at_ref, w1d_ref, c1_ref, w2_ref, b2_ref,
               wx1_ref, bx1c_ref, wx2c_ref, bx2_ref, m_ref, p_ref):
    drterm = lax.dot_general(dr_ref[...], c1_ref[...],
                             (((0,), (0,)), ((), ())),
                             preferred_element_type=jnp.float32)
    u = jnp.tanh(pre_ref[...] + drterm
                 + jnp.dot(at_ref[...], w1d_ref[...], preferred_element_type=jnp.float32))
    m = jnp.tanh(jnp.dot(u.astype(jnp.bfloat16),
                         w2_ref[...].astype(jnp.bfloat16),
                         preferred_element_type=jnp.float32) + b2_ref[...])
    m_ref[...] = m
    t_t = jnp.tanh(lax.dot_general(wx1_ref[...].astype(jnp.bfloat16),
                                   m.astype(jnp.bfloat16),
                                   (((1,), (1,)), ((), ())),
                                   preferred_element_type=jnp.float32)
                   + bx1c_ref[...])
    s = jnp.sum(t_t * wx2c_ref[...], axis=0, keepdims=True) + bx2_ref[...]
    p_ref[...] = jnp.tanh(s)


def _edge_mlp(pre, dr2, attrs, we1d_t, c1, we2_t, be2, wx1, bx1c, wx2c, bx2, be):
    e, d = pre.shape
    ij = attrs.shape[1]
    return pl.pallas_call(
        _edge_body,
        grid=(e // be,),
        in_specs=[
            pl.BlockSpec((be, d), lambda i: (i, 0)),
            pl.BlockSpec((1, be), lambda i: (0, i)),
            pl.BlockSpec((be, ij), lambda i: (i, 0)),
            pl.BlockSpec((ij, d), lambda i: (0, 0)),
            pl.BlockSpec((1, d), lambda i: (0, 0)),
            pl.BlockSpec((d, d), lambda i: (0, 0)),
            pl.BlockSpec((1, d), lambda i: (0, 0)),
            pl.BlockSpec((d, d), lambda i: (0, 0)),
            pl.BlockSpec((d, 1), lambda i: (0, 0)),
            pl.BlockSpec((d, 1), lambda i: (0, 0)),
            pl.BlockSpec((1, 1), lambda i: (0, 0)),
        ],
        out_specs=[
            pl.BlockSpec((be, d), lambda i: (i, 0)),
            pl.BlockSpec((1, be), lambda i: (0, i)),
        ],
        out_shape=[
            jax.ShapeDtypeStruct((e, d), jnp.float32),
            jax.ShapeDtypeStruct((1, e), jnp.float32),
        ],
    )(pre, dr2, attrs, we1d_t, c1, we2_t, be2, wx1, bx1c, wx2c, bx2)


def _sc_scatter_mi(m, lins, e_pad, kch, n_acc, base):
    d = m.shape[1]
    rpt = n_acc // NS
    mesh = plsc.VectorSubcoreMesh(core_axis_name="c", subcore_axis_name="s",
                                  num_cores=1)

    @functools.partial(
        pl.kernel,
        out_type=jax.ShapeDtypeStruct((n_acc, d), jnp.float32),
        mesh=mesh,
        scratch_types=[
            pltpu.VMEM((8, CH), jnp.int32),
            pltpu.VMEM((CH, d), jnp.float32),
            pltpu.VMEM((CH, d), jnp.float32),
            pltpu.VMEM_SHARED((n_acc, d), jnp.float32),
            pltpu.SemaphoreType.DMA,
            pltpu.SemaphoreType.DMA,
        ],
        compiler_params=pltpu.CompilerParams(needs_layout_passes=False),
    )
    def k(m_hbm, lin_hbm, mi_hbm, lin_v, mbuf0, mbuf1, mi_acc, s0, s1):
        sid = lax.axis_index("s")
        row0 = base + sid * kch
        mbuf = (mbuf0, mbuf1)
        sem = (s0, s1)

        @pl.loop(0, CH)
        def _zrow(i):
            for cc in range(d // L):
                mbuf0[i, pl.ds(cc * L, L)] = jnp.zeros((L,), jnp.float32)

        @pl.loop(0, rpt // CH)
        def _zinit(bk):
            pltpu.sync_copy(mbuf0, mi_acc.at[pl.ds(sid * rpt + bk * CH, CH)])

        plsc.subcore_barrier()

        @pl.loop(0, kch // 8)
        def _blk(j8):
            row = row0 + j8 * 8
            rloc = sid * kch + j8 * 8
            pltpu.sync_copy(lin_hbm.at[pl.ds(row, 8)], lin_v)
            waits = [pltpu.async_copy(m_hbm.at[pl.ds(rloc * CH, CH)],
                                      mbuf[0], sem[0])]
            for jj in range(8):
                par = jj & 1
                waits[-1].wait()
                if jj < 7:
                    waits.append(pltpu.async_copy(
                        m_hbm.at[pl.ds((rloc + jj + 1) * CH, CH)],
                        mbuf[1 - par], sem[1 - par]))
                pltpu.sync_copy(mbuf[par], mi_acc.at[lin_v.at[jj]], add=True)

        plsc.subcore_barrier()

        @pl.loop(0, rpt // CH)
        def _dump(bk):
            base = sid * rpt + bk * CH
            pltpu.sync_copy(mi_acc.at[pl.ds(base, CH)], mi_hbm.at[pl.ds(base, CH)])

    return k(m, lins)


def _sc_scatter_nct(p2, lins, colg, xq0, xq1, xq2, e_pad, kch, nt4, base):
    nq = xq0.shape[0]
    mesh = plsc.VectorSubcoreMesh(core_axis_name="c", subcore_axis_name="s",
                                  num_cores=1)

    @functools.partial(
        pl.kernel,
        out_type=jax.ShapeDtypeStruct((NS, nt4), jnp.float32),
        mesh=mesh,
        scratch_types=[
            pltpu.VMEM((8, CH), jnp.int32),
            pltpu.VMEM((8, CH), jnp.int32),
            pltpu.VMEM((CH * 8 + 8,), jnp.float32),
            pltpu.VMEM((1, CH), jnp.float32),
            pltpu.VMEM((1, CH), jnp.float32),
            pltpu.VMEM((nq,), jnp.float32),
            pltpu.VMEM((nq,), jnp.float32),
            pltpu.VMEM((nq,), jnp.float32),
            pltpu.VMEM((nt4,), jnp.float32),
            pltpu.SemaphoreType.DMA,
            pltpu.SemaphoreType.DMA,
        ],
        compiler_params=pltpu.CompilerParams(needs_layout_passes=False),
    )
    def k(p_hbm, lin_hbm, col_hbm, x0_hbm, x1_hbm, x2_hbm, nct_hbm,
          linw, colw, dqbuf, pbuf0, pbuf1, x0v, x1v, x2v, nct, s0, s1):
        sid = lax.axis_index("s")
        row0 = base + sid * kch
        rloc0 = sid * kch
        pbuf = (pbuf0, pbuf1)
        sem = (s0, s1)
        pltpu.sync_copy(x0_hbm, x0v)
        pltpu.sync_copy(x1_hbm, x1v)
        pltpu.sync_copy(x2_hbm, x2v)

        @pl.loop(0, nt4, step=L)
        def _znct(i):
            nct[pl.ds(i, L)] = jnp.zeros((L,), jnp.float32)

        @pl.loop(0, CH * 8 + 8, step=L)
        def _zdq(i):
            dqbuf[pl.ds(i, L)] = jnp.zeros((L,), jnp.float32)

        iota = lax.iota(jnp.int32, L)
        lane4 = iota < 4
        one3 = jnp.where(iota == 3, 1.0, 0.0).astype(jnp.float32)

        pltpu.async_copy(p_hbm.at[:, pl.ds(rloc0 * CH, CH)], pbuf[0], sem[0])

        @pl.loop(0, kch, step=8)
        def _blk(j8):
            row = row0 + j8
            pltpu.sync_copy(lin_hbm.at[pl.ds(row, 8)], linw)
            pltpu.sync_copy(col_hbm.at[pl.ds(row, 8)], colw)
            for jj in range(8):
                par = jj & 1
                c = j8 + jj
                pltpu.make_async_copy(p_hbm.at[:, pl.ds(0, CH)], pbuf[par],
                                      sem[par]).wait()

                @pl.when(c + 1 < kch)
                def _():
                    pltpu.async_copy(
                        p_hbm.at[:, pl.ds((rloc0 + c + 1) * CH, CH)],
                        pbuf[1 - par], sem[1 - par])

                for g in range(CH // L):
                    sl = pl.ds(g * L, L)
                    li = linw[jj, sl]
                    ci = colw[jj, sl]
                    base = iota * 8 + (g * L * 8)
                    dx = plsc.load_gather(x0v, [li]) - plsc.load_gather(x0v, [ci])
                    dy = plsc.load_gather(x1v, [li]) - plsc.load_gather(x1v, [ci])
                    dz = plsc.load_gather(x2v, [li]) - plsc.load_gather(x2v, [ci])
                    plsc.store_scatter(dqbuf, [base], dx)
                    plsc.store_scatter(dqbuf, [base + 1], dy)
                    plsc.store_scatter(dqbuf, [base + 2], dz)

                pb = pbuf[par]

                @pl.loop(0, CH // L)
                def _g(g):
                    li = linw[jj, pl.ds(g * L, L)]
                    p16 = pb[0, pl.ds(g * L, L)]
                    for gg in range(L):
                        node = li[gg]
                        ps = p16[gg]
                        dqrow = dqbuf[pl.ds((g * L + gg) * 8, L)]
                        wrow = dqrow * ps + one3
                        plsc.addupdate_scatter(nct, [iota + node * 4], wrow,
                                               mask=lane4)

        pltpu.sync_copy(nct, nct_hbm.at[sid])

    return k(p2, lins, colg, xq0, xq1, xq2)


def _node_upd_body(h_ref, mi0_ref, mi1_ref, mi2_ref, mi3_ref,
                   nc_ref, xp_ref, vp_ref,
                   wv1_ref, bv1_ref, wv2_ref, bv2_ref,
                   wh1a_ref, wh1b_ref, bh1_ref, wh2_ref, bh2_ref,
                   hn_ref, xn_ref, vn_ref):
    h = h_ref[...]
    mi = (mi0_ref[...] + mi1_ref[...]) + (mi2_ref[...] + mi3_ref[...])
    nc = nc_ref[...]
    tv = jnp.tanh(jnp.dot(h, wv1_ref[...], preferred_element_type=jnp.float32) + bv1_ref[...])
    phi_v = jnp.sum(tv * wv2_ref[...], axis=1, keepdims=True) + bv2_ref[...]
    cnt = nc[:, 3:4]
    media = nc / jnp.maximum(cnt, 1.0)
    vn = vp_ref[...] * phi_v + media
    vn_ref[...] = vn
    xn_ref[...] = xp_ref[...] + vn
    hu = jnp.tanh(jnp.dot(h, wh1a_ref[...], preferred_element_type=jnp.float32)
                  + jnp.dot(mi, wh1b_ref[...], preferred_element_type=jnp.float32)
                  + bh1_ref[...])
    hn_ref[...] = jnp.dot(hu, wh2_ref[...], preferred_element_type=jnp.float32) + bh2_ref[...]


def _node_update(h, mis, nc16, xp, vp, wv1_t, bv1, wv2, bv2,
                 wh1a_t, wh1b_t, bh1, wh2_t, bh2, bn):
    n, d = h.shape
    so = wh2_t.shape[1]
    return pl.pallas_call(
        _node_upd_body,
        grid=(n // bn,),
        in_specs=[
            pl.BlockSpec((bn, d), lambda i: (i, 0)),
            pl.BlockSpec((bn, d), lambda i: (i, 0)),
            pl.BlockSpec((bn, d), lambda i: (i, 0)),
            pl.BlockSpec((bn, d), lambda i: (i, 0)),
            pl.BlockSpec((bn, d), lambda i: (i, 0)),
            pl.BlockSpec((bn, 16), lambda i: (i, 0)),
            pl.BlockSpec((bn, 16), lambda i: (i, 0)),
            pl.BlockSpec((bn, 16), lambda i: (i, 0)),
            pl.BlockSpec((d, d), lambda i: (0, 0)),
            pl.BlockSpec((1, d), lambda i: (0, 0)),
            pl.BlockSpec((1, d), lambda i: (0, 0)),
            pl.BlockSpec((1, 1), lambda i: (0, 0)),
            pl.BlockSpec((d, d), lambda i: (0, 0)),
            pl.BlockSpec((d, d), lambda i: (0, 0)),
            pl.BlockSpec((1, d), lambda i: (0, 0)),
            pl.BlockSpec((d, so), lambda i: (0, 0)),
            pl.BlockSpec((1, so), lambda i: (0, 0)),
        ],
        out_specs=[
            pl.BlockSpec((bn, so), lambda i: (i, 0)),
            pl.BlockSpec((bn, 16), lambda i: (i, 0)),
            pl.BlockSpec((bn, 16), lambda i: (i, 0)),
        ],
        out_shape=[
            jax.ShapeDtypeStruct((n, so), jnp.float32),
            jax.ShapeDtypeStruct((n, 16), jnp.float32),
            jax.ShapeDtypeStruct((n, 16), jnp.float32),
        ],
    )(h, mis[0], mis[1], mis[2], mis[3], nc16, xp, vp,
      wv1_t, bv1, wv2, bv2, wh1a_t, wh1b_t, bh1, wh2_t, bh2)


def kernel(h, x, arestas, velocidade, atributos_arestas,
           We1, be1, We2, be2, Wx1, bx1, Wx2, bx2,
           Wv1, bv1, Wv2, bv2, Wh1, bh1, Wh2, bh2):
    n, ent = h.shape
    e = arestas.shape[1]
    lin = arestas[0]
    col = arestas[1]

    kch_g = -(-e // (NC * NS * CH))
    kch_g = -(-kch_g // 8) * 8
    e_pad = NC * NS * kch_g * CH
    kch_s = e_pad // (NS * CH)
    n_acc = -(-(n + 1) // (NS * CH)) * NS * CH
    nt4 = -(-(4 * (n + 1)) // 128) * 128

    we1a_t = We1[:, :ent].T
    we1b_t = We1[:, ent:2 * ent].T
    c1 = We1[:, 2 * ent].reshape(1, -1)
    we1d_t = We1[:, 2 * ent + 1:].T
    be1r = be1.reshape(1, -1)

    a, b = _node_pre(h, we1a_t, we1b_t, be1r, bn=2000)

    pad = e_pad - e
    ling = jnp.pad(lin, (0, pad))
    colg = jnp.pad(col, (0, pad))
    colg2 = colg.reshape(-1, CH)
    lins = jnp.pad(lin, (0, pad), constant_values=n).reshape(-1, CH)
    xq0 = jnp.pad(x[:, 0], (0, 8))
    xq1 = jnp.pad(x[:, 1], (0, 8))
    xq2 = jnp.pad(x[:, 2], (0, 8))

    attrs_p = jnp.pad(atributos_arestas, ((0, pad), (0, 0)))
    nsl = 4
    e4 = e_pad // nsl
    rows4 = e4 // CH
    kch_g4 = kch_g // nsl
    kch_s4 = kch_s // nsl
    mis = []
    ncts = []
    for hh in range(nsl):
        base = hh * rows4
        pre_h, drf_h = _sc_gather(a, b, x[:, 0], x[:, 1], x[:, 2], ling, colg,
                                  e4, kch_g4, base)
        m_h, p_h = _edge_mlp(pre_h, drf_h.reshape(1, e4),
                             attrs_p[hh * e4:(hh + 1) * e4],
                             we1d_t, c1, We2.T, be2.reshape(1, -1),
                             Wx1, bx1.reshape(-1, 1), Wx2.reshape(-1, 1),
                             bx2.reshape(1, 1), be=4096)
        mis.append(_sc_scatter_mi(m_h, lins, e4, kch_s4, n_acc, base))
        ncts.append(_sc_scatter_nct(p_h, lins, colg2, xq0, xq1, xq2,
                                    e4, kch_s4, nt4, base))

    nc16 = jnp.pad(
        ((ncts[0] + ncts[1]) + (ncts[2] + ncts[3])).sum(axis=0)[:4 * n]
        .reshape(n, 4), ((0, 0), (0, 12)))

    xpad = jnp.pad(x, ((0, 0), (0, 13)))
    vp = jnp.pad(velocidade, ((0, 0), (0, 13)))
    hn, xn, vn = _node_update(
        h, mis, nc16, xpad, vp,
        Wv1.T, bv1.reshape(1, -1), Wv2.reshape(1, -1), bv2.reshape(1, 1),
        Wh1[:, :ent].T, Wh1[:, ent:].T, bh1.reshape(1, -1), Wh2.T,
        bh2.reshape(1, -1), bn=2000)
    return hn, xn[:, :3], vn[:, :3]


# speedup vs baseline: 1.0686x; 1.0686x over previous
import functools

import jax
import jax.numpy as jnp
from jax import lax
from jax.experimental import pallas as pl
from jax.experimental.pallas import tpu as pltpu
from jax.experimental.pallas import tpu_sc as plsc

NC = 2
NS = 16
L = 16
CH = 128


def _node_pre_body(h_ref, wa_ref, wb_ref, ba_ref, a_ref, b_ref):
    h = h_ref[...]
    a_ref[...] = jnp.dot(h, wa_ref[...], preferred_element_type=jnp.float32) + ba_ref[...]
    b_ref[...] = jnp.dot(h, wb_ref[...], preferred_element_type=jnp.float32)


def _node_pre(h, we1a_t, we1b_t, be1, bn):
    n, d = h.shape
    return pl.pallas_call(
        _node_pre_body,
        grid=(n // bn,),
        in_specs=[
            pl.BlockSpec((bn, d), lambda i: (i, 0)),
            pl.BlockSpec((d, d), lambda i: (0, 0)),
            pl.BlockSpec((d, d), lambda i: (0, 0)),
            pl.BlockSpec((1, d), lambda i: (0, 0)),
        ],
        out_specs=[
            pl.BlockSpec((bn, d), lambda i: (i, 0)),
            pl.BlockSpec((bn, d), lambda i: (i, 0)),
        ],
        out_shape=[
            jax.ShapeDtypeStruct((n, d), jnp.float32),
            jax.ShapeDtypeStruct((n, d), jnp.float32),
        ],
    )(h, we1a_t, we1b_t, be1)


def _sc_gather(a, b, x0, x1, x2, ling, colg, e_pad, kch, base):
    n, d = a.shape
    mesh = plsc.VectorSubcoreMesh(core_axis_name="c", subcore_axis_name="s")

    @functools.partial(
        pl.kernel,
        out_type=[
            jax.ShapeDtypeStruct((e_pad, d), jnp.float32),
            jax.ShapeDtypeStruct((e_pad,), jnp.float32),
        ],
        mesh=mesh,
        scratch_types=[
            pltpu.VMEM((kch * CH,), jnp.int32),
            pltpu.VMEM((kch * CH,), jnp.int32),
            pltpu.VMEM((CH, d), jnp.float32),
            pltpu.VMEM((CH, d), jnp.float32),
            pltpu.VMEM((CH, d), jnp.float32),
            pltpu.VMEM((CH, d), jnp.float32),
            pltpu.VMEM((CH,), jnp.float32),
            pltpu.VMEM((CH,), jnp.float32),
            pltpu.VMEM((n,), jnp.float32),
            pltpu.VMEM((n,), jnp.float32),
            pltpu.VMEM((n,), jnp.float32),
            pltpu.SemaphoreType.DMA,
            pltpu.SemaphoreType.DMA,
            pltpu.SemaphoreType.DMA,
            pltpu.SemaphoreType.DMA,
        ],
        compiler_params=pltpu.CompilerParams(needs_layout_passes=False),
    )
    def k(a_hbm, b_hbm, x0_hbm, x1_hbm, x2_hbm, lin_hbm, col_hbm,
          pre_hbm, drf_hbm,
          linw, colw, bufa0, bufa1, bufb0, bufb1,
          drb0, drb1, x0v, x1v, x2v, si0, si1, so0, so1):
        wid = lax.axis_index("s") * NC + lax.axis_index("c")
        row0 = base + wid * kch
        rloc0 = wid * kch
        bufa = (bufa0, bufa1)
        bufb = (bufb0, bufb1)
        drb = (drb0, drb1)
        s_in = (si0, si1)
        s_out = (so0, so1)
        pltpu.sync_copy(x0_hbm, x0v)
        pltpu.sync_copy(x1_hbm, x1v)
        pltpu.sync_copy(x2_hbm, x2v)

        pltpu.sync_copy(lin_hbm.at[pl.ds(row0 * CH, kch * CH)], linw)
        pltpu.sync_copy(col_hbm.at[pl.ds(row0 * CH, kch * CH)], colw)
        pltpu.async_copy(a_hbm.at[linw.at[pl.ds(0, CH)]], bufa[0], s_in[0])
        pltpu.async_copy(b_hbm.at[colw.at[pl.ds(0, CH)]], bufb[0], s_in[0])

        @pl.loop(0, kch, step=4)
        def _blk(j4):
            for jj in range(4):
                par = jj & 1
                c = j4 + jj
                pltpu.make_async_copy(a_hbm.at[linw.at[pl.ds(0, CH)]],
                                      bufa[par], s_in[par]).wait()
                pltpu.make_async_copy(b_hbm.at[colw.at[pl.ds(0, CH)]],
                                      bufb[par], s_in[par]).wait()

                for g in range(CH // L):
                    sl = pl.ds(c * CH + g * L, L)
                    li = linw[sl]
                    ci = colw[sl]
                    dx = plsc.load_gather(x0v, [li]) - plsc.load_gather(x0v, [ci])
                    dy = plsc.load_gather(x1v, [li]) - plsc.load_gather(x1v, [ci])
                    dz = plsc.load_gather(x2v, [li]) - plsc.load_gather(x2v, [ci])
                    drb[par][pl.ds(g * L, L)] = dx * dx + dy * dy + dz * dz

                @pl.when(c >= 1)
                def _():
                    pltpu.make_async_copy(bufb[1 - par],
                                          pre_hbm.at[pl.ds(0, CH)],
                                          s_out[1 - par]).wait()
                    pltpu.make_async_copy(drb[1 - par],
                                          drf_hbm.at[pl.ds(0, CH)],
                                          s_out[1 - par]).wait()

                @pl.when(c + 1 < kch)
                def _():
                    pltpu.async_copy(
                        a_hbm.at[linw.at[pl.ds((c + 1) * CH, CH)]],
                        bufa[1 - par], s_in[1 - par])
                    pltpu.async_copy(
                        b_hbm.at[colw.at[pl.ds((c + 1) * CH, CH)]],
                        bufb[1 - par], s_in[1 - par])

                ba = bufa[par]
                bb = bufb[par]

                @pl.loop(0, CH)
                def _row(i):
                    for cc in range(d // L):
                        sl = pl.ds(cc * L, L)
                        bb[i, sl] = ba[i, sl] + bb[i, sl]

                r = rloc0 + c
                pltpu.async_copy(bb, pre_hbm.at[pl.ds(r * CH, CH)], s_out[par])
                pltpu.async_copy(drb[par], drf_hbm.at[pl.ds(r * CH, CH)],
                                 s_out[par])

        pltpu.make_async_copy(bufb[1], pre_hbm.at[pl.ds(0, CH)],
                              s_out[1]).wait()
        pltpu.make_async_copy(drb[1], drf_hbm.at[pl.ds(0, CH)],
                              s_out[1]).wait()

    return k(a, b, x0, x1, x2, ling, colg)


def _edge_body(pre_ref, dr_ref, at_ref, w1d_ref, c1_ref, w2_ref, b2_ref,
               wx1_ref, bx1c_ref, wx2c_ref, bx2_ref, m_ref, p_ref):
    drterm = lax.dot_general(dr_ref[...], c1_ref[...],
                             (((0,), (0,)), ((), ())),
                             preferred_element_type=jnp.float32)
    u = jnp.tanh(pre_ref[...] + drterm
                 + jnp.dot(at_ref[...], w1d_ref[...], preferred_element_type=jnp.float32))
    m = jnp.tanh(jnp.dot(u.astype(jnp.bfloat16),
                         w2_ref[...].astype(jnp.bfloat16),
                         preferred_element_type=jnp.float32) + b2_ref[...])
    m_ref[...] = m
    t_t = jnp.tanh(lax.dot_general(wx1_ref[...].astype(jnp.bfloat16),
                                   m.astype(jnp.bfloat16),
                                   (((1,), (1,)), ((), ())),
                                   preferred_element_type=jnp.float32)
                   + bx1c_ref[...])
    s = jnp.sum(t_t * wx2c_ref[...], axis=0, keepdims=True) + bx2_ref[...]
    p_ref[...] = jnp.tanh(s)


def _edge_mlp(pre, dr2, attrs, we1d_t, c1, we2_t, be2, wx1, bx1c, wx2c, bx2, be):
    e, d = pre.shape
    ij = attrs.shape[1]
    return pl.pallas_call(
        _edge_body,
        grid=(e // be,),
        in_specs=[
            pl.BlockSpec((be, d), lambda i: (i, 0)),
            pl.BlockSpec((1, be), lambda i: (0, i)),
            pl.BlockSpec((be, ij), lambda i: (i, 0)),
            pl.BlockSpec((ij, d), lambda i: (0, 0)),
            pl.BlockSpec((1, d), lambda i: (0, 0)),
            pl.BlockSpec((d, d), lambda i: (0, 0)),
            pl.BlockSpec((1, d), lambda i: (0, 0)),
            pl.BlockSpec((d, d), lambda i: (0, 0)),
            pl.BlockSpec((d, 1), lambda i: (0, 0)),
            pl.BlockSpec((d, 1), lambda i: (0, 0)),
            pl.BlockSpec((1, 1), lambda i: (0, 0)),
        ],
        out_specs=[
            pl.BlockSpec((be, d), lambda i: (i, 0)),
            pl.BlockSpec((1, be), lambda i: (0, i)),
        ],
        out_shape=[
            jax.ShapeDtypeStruct((e, d), jnp.float32),
            jax.ShapeDtypeStruct((1, e), jnp.float32),
        ],
    )(pre, dr2, attrs, we1d_t, c1, we2_t, be2, wx1, bx1c, wx2c, bx2)


def _sc_scatter_mi(m, lins, e_pad, kch, n_acc, base):
    d = m.shape[1]
    rpt = n_acc // NS
    mesh = plsc.VectorSubcoreMesh(core_axis_name="c", subcore_axis_name="s",
                                  num_cores=1)

    @functools.partial(
        pl.kernel,
        out_type=jax.ShapeDtypeStruct((n_acc, d), jnp.float32),
        mesh=mesh,
        scratch_types=[
            pltpu.VMEM((8, CH), jnp.int32),
            pltpu.VMEM((CH, d), jnp.float32),
            pltpu.VMEM((CH, d), jnp.float32),
            pltpu.VMEM_SHARED((n_acc, d), jnp.float32),
            pltpu.SemaphoreType.DMA,
            pltpu.SemaphoreType.DMA,
        ],
        compiler_params=pltpu.CompilerParams(needs_layout_passes=False),
    )
    def k(m_hbm, lin_hbm, mi_hbm, lin_v, mbuf0, mbuf1, mi_acc, s0, s1):
        sid = lax.axis_index("s")
        row0 = base + sid * kch
        mbuf = (mbuf0, mbuf1)
        sem = (s0, s1)

        @pl.loop(0, CH)
        def _zrow(i):
            for cc in range(d // L):
                mbuf0[i, pl.ds(cc * L, L)] = jnp.zeros((L,), jnp.float32)

        @pl.loop(0, rpt // CH)
        def _zinit(bk):
            pltpu.sync_copy(mbuf0, mi_acc.at[pl.ds(sid * rpt + bk * CH, CH)])

        plsc.subcore_barrier()

        @pl.loop(0, kch // 8)
        def _blk(j8):
            row = row0 + j8 * 8
            rloc = sid * kch + j8 * 8
            pltpu.sync_copy(lin_hbm.at[pl.ds(row, 8)], lin_v)
            waits = [pltpu.async_copy(m_hbm.at[pl.ds(rloc * CH, CH)],
                                      mbuf[0], sem[0])]
            for jj in range(8):
                par = jj & 1
                waits[-1].wait()
                if jj < 7:
                    waits.append(pltpu.async_copy(
                        m_hbm.at[pl.ds((rloc + jj + 1) * CH, CH)],
                        mbuf[1 - par], sem[1 - par]))
                pltpu.sync_copy(mbuf[par], mi_acc.at[lin_v.at[jj]], add=True)

        plsc.subcore_barrier()

        @pl.loop(0, rpt // CH)
        def _dump(bk):
            base = sid * rpt + bk * CH
            pltpu.sync_copy(mi_acc.at[pl.ds(base, CH)], mi_hbm.at[pl.ds(base, CH)])

    return k(m, lins)


def _sc_scatter_nct(p2, lins, colg, xq0, xq1, xq2, e_pad, kch, nt4, base):
    nq = xq0.shape[0]
    mesh = plsc.VectorSubcoreMesh(core_axis_name="c", subcore_axis_name="s",
                                  num_cores=1)

    @functools.partial(
        pl.kernel,
        out_type=jax.ShapeDtypeStruct((NS, nt4), jnp.float32),
        mesh=mesh,
        scratch_types=[
            pltpu.VMEM((8, CH), jnp.int32),
            pltpu.VMEM((8, CH), jnp.int32),
            pltpu.VMEM((CH * 8 + 8,), jnp.float32),
            pltpu.VMEM((1, CH), jnp.float32),
            pltpu.VMEM((1, CH), jnp.float32),
            pltpu.VMEM((nq,), jnp.float32),
            pltpu.VMEM((nq,), jnp.float32),
            pltpu.VMEM((nq,), jnp.float32),
            pltpu.VMEM((nt4,), jnp.float32),
            pltpu.SemaphoreType.DMA,
            pltpu.SemaphoreType.DMA,
        ],
        compiler_params=pltpu.CompilerParams(needs_layout_passes=False),
    )
    def k(p_hbm, lin_hbm, col_hbm, x0_hbm, x1_hbm, x2_hbm, nct_hbm,
          linw, colw, dqbuf, pbuf0, pbuf1, x0v, x1v, x2v, nct, s0, s1):
        sid = lax.axis_index("s")
        row0 = base + sid * kch
        rloc0 = sid * kch
        pbuf = (pbuf0, pbuf1)
        sem = (s0, s1)
        pltpu.sync_copy(x0_hbm, x0v)
        pltpu.sync_copy(x1_hbm, x1v)
        pltpu.sync_copy(x2_hbm, x2v)

        @pl.loop(0, nt4, step=L)
        def _znct(i):
            nct[pl.ds(i, L)] = jnp.zeros((L,), jnp.float32)

        @pl.loop(0, CH * 8 + 8, step=L)
        def _zdq(i):
            dqbuf[pl.ds(i, L)] = jnp.zeros((L,), jnp.float32)

        iota = lax.iota(jnp.int32, L)
        lane4 = iota < 4
        one3 = jnp.where(iota == 3, 1.0, 0.0).astype(jnp.float32)

        pltpu.async_copy(p_hbm.at[:, pl.ds(rloc0 * CH, CH)], pbuf[0], sem[0])

        @pl.loop(0, kch, step=8)
        def _blk(j8):
            row = row0 + j8
            pltpu.sync_copy(lin_hbm.at[pl.ds(row, 8)], linw)
            pltpu.sync_copy(col_hbm.at[pl.ds(row, 8)], colw)
            for jj in range(8):
                par = jj & 1
                c = j8 + jj
                pltpu.make_async_copy(p_hbm.at[:, pl.ds(0, CH)], pbuf[par],
                                      sem[par]).wait()

                @pl.when(c + 1 < kch)
                def _():
                    pltpu.async_copy(
                        p_hbm.at[:, pl.ds((rloc0 + c + 1) * CH, CH)],
                        pbuf[1 - par], sem[1 - par])

                for g in range(CH // L):
                    sl = pl.ds(g * L, L)
                    li = linw[jj, sl]
                    ci = colw[jj, sl]
                    base = iota * 8 + (g * L * 8)
                    dx = plsc.load_gather(x0v, [li]) - plsc.load_gather(x0v, [ci])
                    dy = plsc.load_gather(x1v, [li]) - plsc.load_gather(x1v, [ci])
                    dz = plsc.load_gather(x2v, [li]) - plsc.load_gather(x2v, [ci])
                    plsc.store_scatter(dqbuf, [base], dx)
                    plsc.store_scatter(dqbuf, [base + 1], dy)
                    plsc.store_scatter(dqbuf, [base + 2], dz)

                pb = pbuf[par]

                @pl.loop(0, CH // L)
                def _g(g):
                    li = linw[jj, pl.ds(g * L, L)]
                    p16 = pb[0, pl.ds(g * L, L)]
                    for gg in range(L):
                        node = li[gg]
                        ps = p16[gg]
                        dqrow = dqbuf[pl.ds((g * L + gg) * 8, L)]
                        wrow = dqrow * ps + one3
                        plsc.addupdate_scatter(nct, [iota + node * 4], wrow,
                                               mask=lane4)

        pltpu.sync_copy(nct, nct_hbm.at[sid])

    return k(p2, lins, colg, xq0, xq1, xq2)


def _node_upd_body(h_ref, mi0_ref, mi1_ref, mi2_ref, mi3_ref,
                   nc_ref, xp_ref, vp_ref,
                   wv1_ref, bv1_ref, wv2_ref, bv2_ref,
                   wh1a_ref, wh1b_ref, bh1_ref, wh2_ref, bh2_ref,
                   hn_ref, xn_ref, vn_ref):
    h = h_ref[...]
    mi = (mi0_ref[...] + mi1_ref[...]) + (mi2_ref[...] + mi3_ref[...])
    nc = nc_ref[...]
    tv = jnp.tanh(jnp.dot(h, wv1_ref[...], preferred_element_type=jnp.float32) + bv1_ref[...])
    phi_v = jnp.sum(tv * wv2_ref[...], axis=1, keepdims=True) + bv2_ref[...]
    cnt = nc[:, 3:4]
    media = nc / jnp.maximum(cnt, 1.0)
    vn = vp_ref[...] * phi_v + media
    vn_ref[...] = vn
    xn_ref[...] = xp_ref[...] + vn
    hu = jnp.tanh(jnp.dot(h, wh1a_ref[...], preferred_element_type=jnp.float32)
                  + jnp.dot(mi, wh1b_ref[...], preferred_element_type=jnp.float32)
                  + bh1_ref[...])
    hn_ref[...] = jnp.dot(hu, wh2_ref[...], preferred_element_type=jnp.float32) + bh2_ref[...]


def _node_update(h, mis, nc16, xp, vp, wv1_t, bv1, wv2, bv2,
                 wh1a_t, wh1b_t, bh1, wh2_t, bh2, bn):
    n, d = h.shape
    so = wh2_t.shape[1]
    return pl.pallas_call(
        _node_upd_body,
        grid=(n // bn,),
        in_specs=[
            pl.BlockSpec((bn, d), lambda i: (i, 0)),
            pl.BlockSpec((bn, d), lambda i: (i, 0)),
            pl.BlockSpec((bn, d), lambda i: (i, 0)),
            pl.BlockSpec((bn, d), lambda i: (i, 0)),
            pl.BlockSpec((bn, d), lambda i: (i, 0)),
            pl.BlockSpec((bn, 16), lambda i: (i, 0)),
            pl.BlockSpec((bn, 16), lambda i: (i, 0)),
            pl.BlockSpec((bn, 16), lambda i: (i, 0)),
            pl.BlockSpec((d, d), lambda i: (0, 0)),
            pl.BlockSpec((1, d), lambda i: (0, 0)),
            pl.BlockSpec((1, d), lambda i: (0, 0)),
            pl.BlockSpec((1, 1), lambda i: (0, 0)),
            pl.BlockSpec((d, d), lambda i: (0, 0)),
            pl.BlockSpec((d, d), lambda i: (0, 0)),
            pl.BlockSpec((1, d), lambda i: (0, 0)),
            pl.BlockSpec((d, so), lambda i: (0, 0)),
            pl.BlockSpec((1, so), lambda i: (0, 0)),
        ],
        out_specs=[
            pl.BlockSpec((bn, so), lambda i: (i, 0)),
            pl.BlockSpec((bn, 16), lambda i: (i, 0)),
            pl.BlockSpec((bn, 16), lambda i: (i, 0)),
        ],
        out_shape=[
            jax.ShapeDtypeStruct((n, so), jnp.float32),
            jax.ShapeDtypeStruct((n, 16), jnp.float32),
            jax.ShapeDtypeStruct((n, 16), jnp.float32),
        ],
    )(h, mis[0], mis[1], mis[2], mis[3], nc16, xp, vp,
      wv1_t, bv1, wv2, bv2, wh1a_t, wh1b_t, bh1, wh2_t, bh2)


def kernel(h, x, arestas, velocidade, atributos_arestas,
           We1, be1, We2, be2, Wx1, bx1, Wx2, bx2,
           Wv1, bv1, Wv2, bv2, Wh1, bh1, Wh2, bh2):
    n, ent = h.shape
    e = arestas.shape[1]
    lin = arestas[0]
    col = arestas[1]

    kch_g = -(-e // (NC * NS * CH))
    kch_g = -(-kch_g // 8) * 8
    e_pad = NC * NS * kch_g * CH
    kch_s = e_pad // (NS * CH)
    n_acc = -(-(n + 1) // (NS * CH)) * NS * CH
    nt4 = -(-(4 * (n + 1)) // 128) * 128

    we1a_t = We1[:, :ent].T
    we1b_t = We1[:, ent:2 * ent].T
    c1 = We1[:, 2 * ent].reshape(1, -1)
    we1d_t = We1[:, 2 * ent + 1:].T
    be1r = be1.reshape(1, -1)

    a, b = _node_pre(h, we1a_t, we1b_t, be1r, bn=2000)

    pad = e_pad - e
    ling = jnp.pad(lin, (0, pad))
    colg = jnp.pad(col, (0, pad))
    colg2 = colg.reshape(-1, CH)
    lins = jnp.pad(lin, (0, pad), constant_values=n).reshape(-1, CH)
    xq0 = jnp.pad(x[:, 0], (0, 8))
    xq1 = jnp.pad(x[:, 1], (0, 8))
    xq2 = jnp.pad(x[:, 2], (0, 8))

    attrs_p = jnp.pad(atributos_arestas, ((0, pad), (0, 0)))
    nsl = 4
    e4 = e_pad // nsl
    rows4 = e4 // CH
    kch_g4 = kch_g // nsl
    kch_s4 = kch_s // nsl
    mis = []
    ncts = []
    for hh in range(nsl):
        base = hh * rows4
        pre_h, drf_h = _sc_gather(a, b, x[:, 0], x[:, 1], x[:, 2], ling, colg,
                                  e4, kch_g4, base)
        m_h, p_h = _edge_mlp(pre_h, drf_h.reshape(1, e4),
                             attrs_p[hh * e4:(hh + 1) * e4],
                             we1d_t, c1, We2.T, be2.reshape(1, -1),
                             Wx1, bx1.reshape(-1, 1), Wx2.reshape(-1, 1),
                             bx2.reshape(1, 1), be=4096)
        mis.append(_sc_scatter_mi(m_h, lins, e4, kch_s4, n_acc, base))
        ncts.append(_sc_scatter_nct(p_h, lins, colg2, xq0, xq1, xq2,
                                    e4, kch_s4, nt4, base))

    nc16 = jnp.pad(
        ((ncts[0] + ncts[1]) + (ncts[2] + ncts[3])).sum(axis=0)[:4 * n]
        .reshape(n, 4), ((0, 0), (0, 12)))

    xpad = jnp.pad(x, ((0, 0), (0, 13)))
    vp = jnp.pad(velocidade, ((0, 0), (0, 13)))
    hn, xn, vn = _node_update(
        h, mis, nc16, xpad, vp,
        Wv1.T, bv1.reshape(1, -1), Wv2.reshape(1, -1), bv2.reshape(1, 1),
        Wh1[:, :ent].T, Wh1[:, ent:].T, bh1.reshape(1, -1), Wh2.T,
        bh2.reshape(1, -1), bn=2000)
    return hn, xn[:, :3], vn[:, :3]
